# baseline (speedup 1.0000x reference)
import functools
import math

import jax
import jax.numpy as jnp
from jax import lax
from jax.experimental import pallas as pl
from jax.experimental.pallas import tpu as pltpu
from jax.experimental.pallas import tpu_sc as plsc

TOP_K = 2
CAPACITY_FACTOR = 1.0
MIN_CAPACITY = 1
EPS = 1e-6


def _pack_bf16(x):
    r, d = x.shape
    lo = lax.bitcast_convert_type(x[:, :d // 2].astype(jnp.bfloat16),
                                  jnp.uint16).astype(jnp.uint32)
    hi = lax.bitcast_convert_type(x[:, d // 2:].astype(jnp.bfloat16),
                                  jnp.uint16).astype(jnp.uint32)
    return lax.bitcast_convert_type(lo | (hi << 16), jnp.float32)


def _unpack_bf16(p, dtype=jnp.float32):
    u = lax.bitcast_convert_type(p, jnp.uint32)
    lo = lax.bitcast_convert_type((u & 0xFFFF).astype(jnp.uint16),
                                  jnp.bfloat16).astype(dtype)
    hi = lax.bitcast_convert_type((u >> 16).astype(jnp.uint16),
                                  jnp.bfloat16).astype(dtype)
    return jnp.concatenate([lo, hi], axis=1)


def _router_body(x0_ref, ln_s_ref, ln_b_ref, wr_ref, br_ref,
                 x_ref, s0_ref, s1_ref, w0_ref, w1_ref, cnt_ref,
                 carry_ref, *, tb, e, cap, dummy):
    i = pl.program_id(0)

    @pl.when(i == 0)
    def _():
        carry_ref[...] = jnp.zeros_like(carry_ref)

    xb = x0_ref[...]
    mean = jnp.mean(xb, axis=1, keepdims=True)
    var = jnp.mean((xb - mean) * (xb - mean), axis=1, keepdims=True)
    xn = (xb - mean) * lax.rsqrt(var + EPS) * ln_s_ref[...] + ln_b_ref[...]
    x = jnp.maximum(xn, 0.0)
    x_ref[...] = _pack_bf16(x)

    logits = jnp.dot(x, wr_ref[...], preferred_element_type=jnp.float32)
    logits = logits + br_ref[...]

    lane = lax.broadcasted_iota(jnp.int32, (tb, e), 1)
    big = jnp.int32(e + 1)
    m0 = jnp.max(logits, axis=1, keepdims=True)
    i0 = jnp.min(jnp.where(logits == m0, lane, big), axis=1)
    masked = jnp.where(lane == i0[:, None], -jnp.inf, logits)
    m1 = jnp.max(masked, axis=1, keepdims=True)
    i1 = jnp.min(jnp.where(masked == m1, lane, big), axis=1)

    e1 = jnp.exp(m1[:, 0] - m0[:, 0])
    g0 = 1.0 / (1.0 + e1)
    g1 = e1 / (1.0 + e1)

    oh0 = (lane == i0[:, None]).astype(jnp.float32)
    oh1 = (lane == i1[:, None]).astype(jnp.float32)
    both = oh0 + oh1

    r = lax.broadcasted_iota(jnp.int32, (tb, tb), 0)
    c = lax.broadcasted_iota(jnp.int32, (tb, tb), 1)
    tri = (r > c).astype(jnp.float32)
    prior = jnp.dot(tri, both, preferred_element_type=jnp.float32)
    prior = prior + carry_ref[...]

    pos0 = jnp.sum(prior * oh0, axis=1).astype(jnp.int32)
    pos1 = jnp.sum(prior * oh1, axis=1).astype(jnp.int32)

    carry_new = carry_ref[...] + jnp.sum(both, axis=0, keepdims=True)
    carry_ref[...] = carry_new
    cnt_ref[...] = carry_new.astype(jnp.int32)

    keep0 = pos0 < cap
    keep1 = pos1 < cap
    s0 = jnp.where(keep0, i0 * cap + pos0, dummy)
    s1 = jnp.where(keep1, i1 * cap + pos1, dummy)
    w0 = jnp.where(keep0, g0, 0.0)
    w1 = jnp.where(keep1, g1, 0.0)

    s0_ref[...] = s0
    s1_ref[...] = s1
    w0_ref[...] = jnp.broadcast_to(w0.reshape(tb, 1), (tb, 128))
    w1_ref[...] = jnp.broadcast_to(w1.reshape(tb, 1), (tb, 128))


def _run_router(x0_flat, ln0_scale, ln0_bias, wr, br, *, cap, dummy):
    n, d = x0_flat.shape
    e = wr.shape[1]
    tb = 256
    nb = n // tb
    body = functools.partial(_router_body, tb=tb, e=e, cap=cap, dummy=dummy)
    out_shapes = (
        jax.ShapeDtypeStruct((n, d // 2), jnp.float32),
        jax.ShapeDtypeStruct((n,), jnp.int32),
        jax.ShapeDtypeStruct((n,), jnp.int32),
        jax.ShapeDtypeStruct((n, 128), jnp.float32),
        jax.ShapeDtypeStruct((n, 128), jnp.float32),
        jax.ShapeDtypeStruct((1, e), jnp.int32),
    )
    grid = (nb,)
    in_specs = [
            pl.BlockSpec((tb, d), lambda i: (i, 0)),
            pl.BlockSpec((1, d), lambda i: (0, 0)),
            pl.BlockSpec((1, d), lambda i: (0, 0)),
            pl.BlockSpec((d, e), lambda i: (0, 0)),
            pl.BlockSpec((1, e), lambda i: (0, 0)),
    ]
    out_specs = [
            pl.BlockSpec((tb, d // 2), lambda i: (i, 0)),
            pl.BlockSpec((tb,), lambda i: (i,)),
            pl.BlockSpec((tb,), lambda i: (i,)),
            pl.BlockSpec((tb, 128), lambda i: (i, 0)),
            pl.BlockSpec((tb, 128), lambda i: (i, 0)),
            pl.BlockSpec((1, e), lambda i: (0, 0)),
    ]
    return pl.pallas_call(
        body,
        grid=grid,
        in_specs=in_specs,
        out_specs=out_specs,
        out_shape=out_shapes,
        scratch_shapes=[pltpu.VMEM((1, e), jnp.float32)],
        compiler_params=pltpu.CompilerParams(
            dimension_semantics=("arbitrary",)),
    )(x0_flat, ln0_scale.reshape(1, d), ln0_bias.reshape(1, d), wr,
      br.reshape(1, e))


def _dispatch_sc(x, s0, s1, w0, w1, *, rows_out, dummy):
    n, d = x.shape
    info = plsc.get_sparse_core_info()
    nc, ns = info.num_cores, info.num_subcores
    nw = nc * ns
    tpw = n // nw
    mesh = plsc.VectorSubcoreMesh(core_axis_name="c", subcore_axis_name="s")

    @functools.partial(
        pl.kernel,
        out_type=(jax.ShapeDtypeStruct((rows_out, d), jnp.float32),
                  jax.ShapeDtypeStruct((rows_out, 128), jnp.float32),
                  jax.ShapeDtypeStruct((rows_out, 128), jnp.int32)),
        mesh=mesh,
        scratch_types=[
            pltpu.VMEM((tpw, d), jnp.float32),
            pltpu.VMEM((tpw, 128), jnp.float32),
            pltpu.VMEM((tpw, 128), jnp.float32),
            pltpu.VMEM((tpw, 128), jnp.int32),
            pltpu.VMEM((tpw,), jnp.int32),
            pltpu.VMEM((tpw,), jnp.int32),
            pltpu.SemaphoreType.DMA,
        ],
    )
    def k(x_hbm, s0_hbm, s1_hbm, w0_hbm, w1_hbm, out_hbm, wout_hbm,
          tok_hbm, xrows, w0r, w1r, tokr, s0v, s1v, sem):
        wid = lax.axis_index("s") * nc + lax.axis_index("c")
        base = wid * tpw
        pltpu.sync_copy(x_hbm.at[pl.ds(base, tpw)], xrows)
        pltpu.sync_copy(w0_hbm.at[pl.ds(base, tpw)], w0r)
        pltpu.sync_copy(w1_hbm.at[pl.ds(base, tpw)], w1r)
        pltpu.sync_copy(s0_hbm.at[pl.ds(base, tpw)], s0v)
        pltpu.sync_copy(s1_hbm.at[pl.ds(base, tpw)], s1v)
        my_dummy = jnp.full((16,), dummy, jnp.int32) + wid
        for cchunk in range(tpw // 16):
            sl = pl.ds(cchunk * 16, 16)
            v0 = s0v[sl]
            s0v[sl] = jnp.where(v0 == dummy, my_dummy, v0)
            v1 = s1v[sl]
            s1v[sl] = jnp.where(v1 == dummy, my_dummy, v1)

        def tok_body(j, _):
            row = jnp.zeros((16,), jnp.int32) + (base + j)
            for cc in range(128 // 16):
                tokr[j, pl.ds(cc * 16, 16)] = row
            return 0

        lax.fori_loop(0, tpw, tok_body, 0)
        d0 = pltpu.async_copy(xrows, out_hbm.at[s0v], sem)
        d1 = pltpu.async_copy(w0r, wout_hbm.at[s0v], sem)
        d2 = pltpu.async_copy(tokr, tok_hbm.at[s0v], sem)
        d0.wait()
        d1.wait()
        d2.wait()

        def tok_body2(j, _):
            row = jnp.zeros((16,), jnp.int32) + (base + n + j)
            for cc in range(128 // 16):
                tokr[j, pl.ds(cc * 16, 16)] = row
            return 0

        lax.fori_loop(0, tpw, tok_body2, 0)
        d3 = pltpu.async_copy(xrows, out_hbm.at[s1v], sem)
        d4 = pltpu.async_copy(w1r, wout_hbm.at[s1v], sem)
        d5 = pltpu.async_copy(tokr, tok_hbm.at[s1v], sem)
        d3.wait()
        d4.wait()
        d5.wait()

    return k(x, s0, s1, w0, w1)


def _mlp_body(cnt_ref, ei_ref, ws_ref, w1_ref, b1_ref, lns_ref, lnb_ref,
              w2_ref, b2_ref, y_ref, *, cap, hb):
    i = pl.program_id(0)
    j = pl.program_id(1)
    a = _unpack_bf16(ei_ref[...], jnp.bfloat16)
    h = jnp.dot(a, w1_ref[0].astype(jnp.bfloat16),
                preferred_element_type=jnp.float32)
    h = h + b1_ref[0]
    mean = jnp.mean(h, axis=1, keepdims=True)
    var = jnp.mean((h - mean) * (h - mean), axis=1, keepdims=True)
    h = (h - mean) * lax.rsqrt(var + EPS) * lns_ref[0] + lnb_ref[0]
    h = jnp.maximum(h, 0.0)
    y = jnp.dot(h.astype(jnp.bfloat16), w2_ref[0].astype(jnp.bfloat16),
                preferred_element_type=jnp.float32)
    y = y + b2_ref[0]
    y = y * ws_ref[:, :1]
    filled = jnp.minimum(cnt_ref[i], cap)
    rows = lax.broadcasted_iota(jnp.int32, (hb, 1), 0) + j * hb
    y_ref[...] = _pack_bf16(jnp.where(rows < filled, y, 0.0))


def _run_mlp(ei_full, wslot, w1, b1, lns, lnb, w2, b2, counts, *, cap):
    e, d, h = w1.shape
    ns = 1
    hb = cap // ns
    grid_spec = pltpu.PrefetchScalarGridSpec(
        num_scalar_prefetch=1,
        grid=(e, ns),
        in_specs=[
            pl.BlockSpec((hb, d // 2), lambda i, j, cnt: (i * ns + j, 0)),
            pl.BlockSpec((hb, 128), lambda i, j, cnt: (i * ns + j, 0)),
            pl.BlockSpec((1, d, h), lambda i, j, cnt: (i, 0, 0)),
            pl.BlockSpec((1, 1, h), lambda i, j, cnt: (i, 0, 0)),
            pl.BlockSpec((1, 1, h), lambda i, j, cnt: (i, 0, 0)),
            pl.BlockSpec((1, 1, h), lambda i, j, cnt: (i, 0, 0)),
            pl.BlockSpec((1, h, d), lambda i, j, cnt: (i, 0, 0)),
            pl.BlockSpec((1, 1, d), lambda i, j, cnt: (i, 0, 0)),
        ],
        out_specs=pl.BlockSpec((hb, d // 2),
                               lambda i, j, cnt: (i * ns + j, 0)),
    )
    return pl.pallas_call(
        functools.partial(_mlp_body, cap=cap, hb=hb),
        grid_spec=grid_spec,
        out_shape=jax.ShapeDtypeStruct((e * cap, d // 2), jnp.float32),
        compiler_params=pltpu.CompilerParams(
            dimension_semantics=("arbitrary", "arbitrary")),
    )(counts, ei_full, wslot, w1, b1.reshape(e, 1, h), lns.reshape(e, 1, h),
      lnb.reshape(e, 1, h), w2, b2.reshape(e, 1, d))


def _combine_sc(y, tokslot, counts16, *, n, cap):
    ncap, d = y.shape
    info = plsc.get_sparse_core_info()
    nc, ns = info.num_cores, info.num_subcores
    nw = nc * ns
    rpw = ncap // nw
    mesh = plsc.VectorSubcoreMesh(core_axis_name="c", subcore_axis_name="s")
    wpe = cap // rpw

    @functools.partial(
        pl.kernel,
        out_type=jax.ShapeDtypeStruct((2 * n + nw, d), jnp.float32),
        mesh=mesh,
        scratch_types=[
            pltpu.VMEM((rpw, d), jnp.float32),
            pltpu.VMEM((rpw, 128), jnp.int32),
            pltpu.VMEM((rpw,), jnp.int32),
            pltpu.VMEM((16, 16), jnp.int32),
            pltpu.SemaphoreType.DMA,
        ],
    )
    def k(y_hbm, tok_hbm, cnt_hbm, yg_hbm, ybuf, tokb, dstv, cntv, sem):
        wid = lax.axis_index("s") * nc + lax.axis_index("c")
        base = wid * rpw
        eid = wid // wpe
        pltpu.sync_copy(cnt_hbm, cntv)
        pltpu.sync_copy(y_hbm.at[pl.ds(base, rpw)], ybuf)
        pltpu.sync_copy(tok_hbm.at[pl.ds(base, rpw)], tokb)
        lanes = lax.iota(jnp.int32, 16)
        filled = jnp.minimum(cntv[eid], cap)
        trash = jnp.full((16,), 2 * n, jnp.int32) + wid
        for cc in range(rpw // 16):
            codes = jnp.zeros((16,), jnp.int32)
            for i in range(16):
                codes = jnp.where(lanes == i, tokb[cc * 16 + i, pl.ds(0, 16)],
                                  codes)
            pos = (base - eid * cap + cc * 16) + lanes
            valid = pos < filled
            dstv[pl.ds(cc * 16, 16)] = jnp.where(valid, codes, trash)
        pltpu.async_copy(ybuf, yg_hbm.at[dstv], sem).wait()

    return k(y, tokslot, counts16)


def _residual_sum_body(x0_ref, y0_ref, y1_ref, w0_ref, w1_ref, out_ref):
    m0 = w0_ref[:, :1] > 0.0
    m1 = w1_ref[:, :1] > 0.0
    y0 = jnp.where(m0, _unpack_bf16(y0_ref[...]), 0.0)
    y1 = jnp.where(m1, _unpack_bf16(y1_ref[...]), 0.0)
    out_ref[...] = x0_ref[...] + y0 + y1


def _run_residual_sum(x0_flat, yg, w0, w1):
    n, d = x0_flat.shape
    tb = 256
    spec = pl.BlockSpec((tb, d), lambda i: (i, 0))
    wspec = pl.BlockSpec((tb, 128), lambda i: (i, 0))
    nb = n // tb
    return pl.pallas_call(
        _residual_sum_body,
        grid=(nb,),
        in_specs=[
            spec,
            pl.BlockSpec((tb, d // 2), lambda i: (i, 0)),
            pl.BlockSpec((tb, d // 2), lambda i: (i + nb, 0)),
            wspec,
            wspec,
        ],
        out_specs=spec,
        out_shape=jax.ShapeDtypeStruct((n, d), jnp.float32),
        compiler_params=pltpu.CompilerParams(
            dimension_semantics=("arbitrary",)),
    )(x0_flat, yg, yg, w0, w1)


def kernel(x0, ln0_scale, ln0_bias, Wr, br, W1, b1, ln1_scale, ln1_bias,
           W2, b2):
    B, S, D = x0.shape
    N = B * S
    E = Wr.shape[-1]
    cap = max(MIN_CAPACITY, int(math.ceil(CAPACITY_FACTOR * N * TOP_K / E)))
    ncap = E * cap
    dummy = ncap
    nw = 32
    rows_out = ncap + nw

    x0_flat = x0.reshape(N, D)
    x, s0, s1, w0_3d, w1_3d, counts = _run_router(
        x0_flat, ln0_scale, ln0_bias, Wr, br, cap=cap, dummy=dummy)

    ei_full, wslot, tokslot = _dispatch_sc(x, s0, s1, w0_3d, w1_3d,
                                           rows_out=rows_out, dummy=dummy)
    y = _run_mlp(ei_full, wslot, W1, b1, ln1_scale, ln1_bias, W2, b2,
                 counts.reshape(E), cap=cap)
    counts16 = jnp.concatenate(
        [counts.reshape(E), jnp.zeros(16 - E, jnp.int32)])
    counts2d = jnp.broadcast_to(counts16[:, None], (16, 16))
    yg = _combine_sc(y, tokslot, counts2d, n=N, cap=cap)
    out = _run_residual_sum(x0_flat, yg, w0_3d, w1_3d)
    return out.reshape(B, S, D)


# speedup vs baseline: 1.0030x; 1.0030x over previous
import functools
import math

import jax
import jax.numpy as jnp
from jax import lax
from jax.experimental import pallas as pl
from jax.experimental.pallas import tpu as pltpu
from jax.experimental.pallas import tpu_sc as plsc

TOP_K = 2
CAPACITY_FACTOR = 1.0
MIN_CAPACITY = 1
EPS = 1e-6


def _pack_bf16(x):
    r, d = x.shape
    lo = lax.bitcast_convert_type(x[:, :d // 2].astype(jnp.bfloat16),
                                  jnp.uint16).astype(jnp.uint32)
    hi = lax.bitcast_convert_type(x[:, d // 2:].astype(jnp.bfloat16),
                                  jnp.uint16).astype(jnp.uint32)
    return lax.bitcast_convert_type(lo | (hi << 16), jnp.float32)


def _unpack_bf16(p, dtype=jnp.float32):
    u = lax.bitcast_convert_type(p, jnp.uint32)
    lo = lax.bitcast_convert_type((u & 0xFFFF).astype(jnp.uint16),
                                  jnp.bfloat16).astype(dtype)
    hi = lax.bitcast_convert_type((u >> 16).astype(jnp.uint16),
                                  jnp.bfloat16).astype(dtype)
    return jnp.concatenate([lo, hi], axis=1)


def _router_body(x0_ref, ln_s_ref, ln_b_ref, wr_ref, br_ref,
                 x_ref, s0_ref, s1_ref, w0_ref, w1_ref, cnt_ref,
                 carry_ref, *, tb, e, cap, dummy):
    i = pl.program_id(0)

    @pl.when(i == 0)
    def _():
        carry_ref[...] = jnp.zeros_like(carry_ref)

    xb = x0_ref[0]
    mean = jnp.mean(xb, axis=1, keepdims=True)
    var = jnp.mean((xb - mean) * (xb - mean), axis=1, keepdims=True)
    xn = (xb - mean) * lax.rsqrt(var + EPS) * ln_s_ref[...] + ln_b_ref[...]
    x = jnp.maximum(xn, 0.0)
    x_ref[...] = _pack_bf16(x)

    logits = jnp.dot(x, wr_ref[...], preferred_element_type=jnp.float32)
    logits = logits + br_ref[...]

    lane = lax.broadcasted_iota(jnp.int32, (tb, e), 1)
    big = jnp.int32(e + 1)
    m0 = jnp.max(logits, axis=1, keepdims=True)
    i0 = jnp.min(jnp.where(logits == m0, lane, big), axis=1)
    masked = jnp.where(lane == i0[:, None], -jnp.inf, logits)
    m1 = jnp.max(masked, axis=1, keepdims=True)
    i1 = jnp.min(jnp.where(masked == m1, lane, big), axis=1)

    e1 = jnp.exp(m1[:, 0] - m0[:, 0])
    g0 = 1.0 / (1.0 + e1)
    g1 = e1 / (1.0 + e1)

    oh0 = (lane == i0[:, None]).astype(jnp.float32)
    oh1 = (lane == i1[:, None]).astype(jnp.float32)
    both = oh0 + oh1

    r = lax.broadcasted_iota(jnp.int32, (tb, tb), 0)
    c = lax.broadcasted_iota(jnp.int32, (tb, tb), 1)
    tri = (r > c).astype(jnp.float32)
    prior = jnp.dot(tri, both, preferred_element_type=jnp.float32)
    prior = prior + carry_ref[...]

    pos0 = jnp.sum(prior * oh0, axis=1).astype(jnp.int32)
    pos1 = jnp.sum(prior * oh1, axis=1).astype(jnp.int32)

    carry_new = carry_ref[...] + jnp.sum(both, axis=0, keepdims=True)
    carry_ref[...] = carry_new
    cnt_ref[...] = carry_new.astype(jnp.int32)

    keep0 = pos0 < cap
    keep1 = pos1 < cap
    s0 = jnp.where(keep0, i0 * cap + pos0, dummy)
    s1 = jnp.where(keep1, i1 * cap + pos1, dummy)
    w0 = jnp.where(keep0, g0, 0.0)
    w1 = jnp.where(keep1, g1, 0.0)

    s0_ref[...] = s0
    s1_ref[...] = s1
    w0_ref[...] = jnp.broadcast_to(w0.reshape(tb, 1), (tb, 128))
    w1_ref[...] = jnp.broadcast_to(w1.reshape(tb, 1), (tb, 128))


def _run_router(x0, ln0_scale, ln0_bias, wr, br, *, cap, dummy):
    b, s, d = x0.shape
    n = b * s
    e = wr.shape[1]
    tb = 256
    nb = n // tb
    body = functools.partial(_router_body, tb=tb, e=e, cap=cap, dummy=dummy)
    out_shapes = (
        jax.ShapeDtypeStruct((n, d // 2), jnp.float32),
        jax.ShapeDtypeStruct((n,), jnp.int32),
        jax.ShapeDtypeStruct((n,), jnp.int32),
        jax.ShapeDtypeStruct((n, 128), jnp.float32),
        jax.ShapeDtypeStruct((n, 128), jnp.float32),
        jax.ShapeDtypeStruct((1, e), jnp.int32),
    )
    grid = (nb,)
    in_specs = [
            pl.BlockSpec((1, tb, d), lambda i: (0, i, 0)),
            pl.BlockSpec((1, d), lambda i: (0, 0)),
            pl.BlockSpec((1, d), lambda i: (0, 0)),
            pl.BlockSpec((d, e), lambda i: (0, 0)),
            pl.BlockSpec((1, e), lambda i: (0, 0)),
    ]
    out_specs = [
            pl.BlockSpec((tb, d // 2), lambda i: (i, 0)),
            pl.BlockSpec((tb,), lambda i: (i,)),
            pl.BlockSpec((tb,), lambda i: (i,)),
            pl.BlockSpec((tb, 128), lambda i: (i, 0)),
            pl.BlockSpec((tb, 128), lambda i: (i, 0)),
            pl.BlockSpec((1, e), lambda i: (0, 0)),
    ]
    return pl.pallas_call(
        body,
        grid=grid,
        in_specs=in_specs,
        out_specs=out_specs,
        out_shape=out_shapes,
        scratch_shapes=[pltpu.VMEM((1, e), jnp.float32)],
        compiler_params=pltpu.CompilerParams(
            dimension_semantics=("arbitrary",)),
    )(x0, ln0_scale.reshape(1, d), ln0_bias.reshape(1, d), wr,
      br.reshape(1, e))


def _dispatch_sc(x, s0, s1, w0, w1, *, rows_out, dummy):
    n, d = x.shape
    info = plsc.get_sparse_core_info()
    nc, ns = info.num_cores, info.num_subcores
    nw = nc * ns
    tpw = n // nw
    mesh = plsc.VectorSubcoreMesh(core_axis_name="c", subcore_axis_name="s")

    @functools.partial(
        pl.kernel,
        out_type=(jax.ShapeDtypeStruct((rows_out, d), jnp.float32),
                  jax.ShapeDtypeStruct((rows_out, 128), jnp.float32),
                  jax.ShapeDtypeStruct((rows_out, 128), jnp.int32)),
        mesh=mesh,
        scratch_types=[
            pltpu.VMEM((tpw, d), jnp.float32),
            pltpu.VMEM((tpw, 128), jnp.float32),
            pltpu.VMEM((tpw, 128), jnp.float32),
            pltpu.VMEM((tpw, 128), jnp.int32),
            pltpu.VMEM((tpw,), jnp.int32),
            pltpu.VMEM((tpw,), jnp.int32),
            pltpu.SemaphoreType.DMA,
        ],
    )
    def k(x_hbm, s0_hbm, s1_hbm, w0_hbm, w1_hbm, out_hbm, wout_hbm,
          tok_hbm, xrows, w0r, w1r, tokr, s0v, s1v, sem):
        wid = lax.axis_index("s") * nc + lax.axis_index("c")
        base = wid * tpw
        pltpu.sync_copy(x_hbm.at[pl.ds(base, tpw)], xrows)
        pltpu.sync_copy(w0_hbm.at[pl.ds(base, tpw)], w0r)
        pltpu.sync_copy(w1_hbm.at[pl.ds(base, tpw)], w1r)
        pltpu.sync_copy(s0_hbm.at[pl.ds(base, tpw)], s0v)
        pltpu.sync_copy(s1_hbm.at[pl.ds(base, tpw)], s1v)
        my_dummy = jnp.full((16,), dummy, jnp.int32) + wid
        for cchunk in range(tpw // 16):
            sl = pl.ds(cchunk * 16, 16)
            v0 = s0v[sl]
            s0v[sl] = jnp.where(v0 == dummy, my_dummy, v0)
            v1 = s1v[sl]
            s1v[sl] = jnp.where(v1 == dummy, my_dummy, v1)

        def tok_body(j, _):
            row = jnp.zeros((16,), jnp.int32) + (base + j)
            for cc in range(128 // 16):
                tokr[j, pl.ds(cc * 16, 16)] = row
            return 0

        lax.fori_loop(0, tpw, tok_body, 0)
        d0 = pltpu.async_copy(xrows, out_hbm.at[s0v], sem)
        d1 = pltpu.async_copy(w0r, wout_hbm.at[s0v], sem)
        d2 = pltpu.async_copy(tokr, tok_hbm.at[s0v], sem)
        d0.wait()
        d1.wait()
        d2.wait()

        def tok_body2(j, _):
            row = jnp.zeros((16,), jnp.int32) + (base + n + j)
            for cc in range(128 // 16):
                tokr[j, pl.ds(cc * 16, 16)] = row
            return 0

        lax.fori_loop(0, tpw, tok_body2, 0)
        d3 = pltpu.async_copy(xrows, out_hbm.at[s1v], sem)
        d4 = pltpu.async_copy(w1r, wout_hbm.at[s1v], sem)
        d5 = pltpu.async_copy(tokr, tok_hbm.at[s1v], sem)
        d3.wait()
        d4.wait()
        d5.wait()

    return k(x, s0, s1, w0, w1)


def _mlp_body(cnt_ref, ei_ref, ws_ref, w1_ref, b1_ref, lns_ref, lnb_ref,
              w2_ref, b2_ref, y_ref, *, cap, hb):
    i = pl.program_id(0)
    j = pl.program_id(1)
    a = _unpack_bf16(ei_ref[...])
    h = jnp.dot(a, w1_ref[0], preferred_element_type=jnp.float32)
    h = h + b1_ref[0]
    mean = jnp.mean(h, axis=1, keepdims=True)
    var = jnp.mean((h - mean) * (h - mean), axis=1, keepdims=True)
    h = (h - mean) * lax.rsqrt(var + EPS) * lns_ref[0] + lnb_ref[0]
    h = jnp.maximum(h, 0.0)
    y = jnp.dot(h, w2_ref[0], preferred_element_type=jnp.float32)
    y = y + b2_ref[0]
    y = y * ws_ref[:, :1]
    filled = jnp.minimum(cnt_ref[i], cap)
    rows = lax.broadcasted_iota(jnp.int32, (hb, 1), 0) + j * hb
    y_ref[...] = _pack_bf16(jnp.where(rows < filled, y, 0.0))


def _run_mlp(ei_full, wslot, w1, b1, lns, lnb, w2, b2, counts, *, cap):
    e, d, h = w1.shape
    ns = 1
    hb = cap // ns
    grid_spec = pltpu.PrefetchScalarGridSpec(
        num_scalar_prefetch=1,
        grid=(e, ns),
        in_specs=[
            pl.BlockSpec((hb, d // 2), lambda i, j, cnt: (i * ns + j, 0)),
            pl.BlockSpec((hb, 128), lambda i, j, cnt: (i * ns + j, 0)),
            pl.BlockSpec((1, d, h), lambda i, j, cnt: (i, 0, 0)),
            pl.BlockSpec((1, 1, h), lambda i, j, cnt: (i, 0, 0)),
            pl.BlockSpec((1, 1, h), lambda i, j, cnt: (i, 0, 0)),
            pl.BlockSpec((1, 1, h), lambda i, j, cnt: (i, 0, 0)),
            pl.BlockSpec((1, h, d), lambda i, j, cnt: (i, 0, 0)),
            pl.BlockSpec((1, 1, d), lambda i, j, cnt: (i, 0, 0)),
        ],
        out_specs=pl.BlockSpec((hb, d // 2),
                               lambda i, j, cnt: (i * ns + j, 0)),
    )
    return pl.pallas_call(
        functools.partial(_mlp_body, cap=cap, hb=hb),
        grid_spec=grid_spec,
        out_shape=jax.ShapeDtypeStruct((e * cap, d // 2), jnp.float32),
        compiler_params=pltpu.CompilerParams(
            dimension_semantics=("arbitrary", "arbitrary")),
    )(counts, ei_full, wslot, w1, b1.reshape(e, 1, h), lns.reshape(e, 1, h),
      lnb.reshape(e, 1, h), w2, b2.reshape(e, 1, d))


def _combine_sc(y, tokslot, counts16, *, n, cap):
    ncap, d = y.shape
    info = plsc.get_sparse_core_info()
    nc, ns = info.num_cores, info.num_subcores
    nw = nc * ns
    rpw = ncap // nw
    mesh = plsc.VectorSubcoreMesh(core_axis_name="c", subcore_axis_name="s")
    wpe = cap // rpw

    @functools.partial(
        pl.kernel,
        out_type=jax.ShapeDtypeStruct((2 * n + nw, d), jnp.float32),
        mesh=mesh,
        scratch_types=[
            pltpu.VMEM((rpw, d), jnp.float32),
            pltpu.VMEM((rpw, 128), jnp.int32),
            pltpu.VMEM((rpw,), jnp.int32),
            pltpu.VMEM((16, 16), jnp.int32),
            pltpu.SemaphoreType.DMA,
        ],
    )
    def k(y_hbm, tok_hbm, cnt_hbm, yg_hbm, ybuf, tokb, dstv, cntv, sem):
        wid = lax.axis_index("s") * nc + lax.axis_index("c")
        base = wid * rpw
        eid = wid // wpe
        pltpu.sync_copy(cnt_hbm, cntv)
        pltpu.sync_copy(y_hbm.at[pl.ds(base, rpw)], ybuf)
        pltpu.sync_copy(tok_hbm.at[pl.ds(base, rpw)], tokb)
        lanes = lax.iota(jnp.int32, 16)
        filled = jnp.minimum(cntv[eid], cap)
        trash = jnp.full((16,), 2 * n, jnp.int32) + wid
        for cc in range(rpw // 16):
            codes = jnp.zeros((16,), jnp.int32)
            for i in range(16):
                codes = jnp.where(lanes == i, tokb[cc * 16 + i, pl.ds(0, 16)],
                                  codes)
            pos = (base - eid * cap + cc * 16) + lanes
            valid = pos < filled
            dstv[pl.ds(cc * 16, 16)] = jnp.where(valid, codes, trash)
        pltpu.async_copy(ybuf, yg_hbm.at[dstv], sem).wait()

    return k(y, tokslot, counts16)


def _residual_sum_body(x0_ref, y0_ref, y1_ref, w0_ref, w1_ref, out_ref):
    m0 = w0_ref[:, :1] > 0.0
    m1 = w1_ref[:, :1] > 0.0
    y0 = jnp.where(m0, _unpack_bf16(y0_ref[...]), 0.0)
    y1 = jnp.where(m1, _unpack_bf16(y1_ref[...]), 0.0)
    out_ref[0] = x0_ref[0] + y0 + y1


def _run_residual_sum(x0, yg, w0, w1):
    b, s, d = x0.shape
    n = b * s
    tb = 256
    spec3 = pl.BlockSpec((1, tb, d), lambda i: (0, i, 0))
    wspec = pl.BlockSpec((tb, 128), lambda i: (i, 0))
    nb = n // tb
    return pl.pallas_call(
        _residual_sum_body,
        grid=(nb,),
        in_specs=[
            spec3,
            pl.BlockSpec((tb, d // 2), lambda i: (i, 0)),
            pl.BlockSpec((tb, d // 2), lambda i: (i + nb, 0)),
            wspec,
            wspec,
        ],
        out_specs=spec3,
        out_shape=jax.ShapeDtypeStruct((b, s, d), jnp.float32),
        compiler_params=pltpu.CompilerParams(
            dimension_semantics=("arbitrary",)),
    )(x0, yg, yg, w0, w1)


def kernel(x0, ln0_scale, ln0_bias, Wr, br, W1, b1, ln1_scale, ln1_bias,
           W2, b2):
    B, S, D = x0.shape
    N = B * S
    E = Wr.shape[-1]
    cap = max(MIN_CAPACITY, int(math.ceil(CAPACITY_FACTOR * N * TOP_K / E)))
    ncap = E * cap
    dummy = ncap
    nw = 32
    rows_out = ncap + nw

    x, s0, s1, w0_3d, w1_3d, counts = _run_router(
        x0, ln0_scale, ln0_bias, Wr, br, cap=cap, dummy=dummy)

    ei_full, wslot, tokslot = _dispatch_sc(x, s0, s1, w0_3d, w1_3d,
                                           rows_out=rows_out, dummy=dummy)
    y = _run_mlp(ei_full, wslot, W1, b1, ln1_scale, ln1_bias, W2, b2,
                 counts.reshape(E), cap=cap)
    counts16 = jnp.concatenate(
        [counts.reshape(E), jnp.zeros(16 - E, jnp.int32)])
    counts2d = jnp.broadcast_to(counts16[:, None], (16, 16))
    yg = _combine_sc(y, tokslot, counts2d, n=N, cap=cap)
    return _run_residual_sum(x0, yg, w0_3d, w1_3d)
